# baseline (speedup 1.0000x reference)
import jax
import jax.numpy as jnp
from jax.experimental import pallas as pl

B_BLK = 64


def _body(x_ref, pos_ref, out_ref):
    out_ref[...] = x_ref[...] + pos_ref[...]


def kernel(x, pos_table):
    B, n, d = x.shape
    x2 = x.reshape(B, n * d)
    pos2 = pos_table[:n].reshape(1, n * d)
    out = pl.pallas_call(
        _body,
        grid=(B // B_BLK,),
        in_specs=[
            pl.BlockSpec((B_BLK, n * d), lambda i: (i, 0)),
            pl.BlockSpec((1, n * d), lambda i: (0, 0)),
        ],
        out_specs=pl.BlockSpec((B_BLK, n * d), lambda i: (i, 0)),
        out_shape=jax.ShapeDtypeStruct((B, n * d), x.dtype),
    )(x2, pos2)
    return out.reshape(B, n, d)
